# Initial kernel scaffold; baseline (speedup 1.0000x reference)
import jax
import jax.numpy as jnp
from jax.experimental import pallas as pl


def kernel(pred, target):
    raise NotImplementedError("write your pallas kernel here")


# speedup vs baseline: 2.2680x; 2.2680x over previous
import functools

import jax
import jax.numpy as jnp
from jax.experimental import pallas as pl
from jax.experimental.pallas import tpu as pltpu

_SMOOTHING = 0.1
_CONFIDENCE = 1.0 - _SMOOTHING


def _nll_block(tgt_ref, x_ref, out_ref, acc_ref, *, n_rows, n_cols, bc, nblk):
    j = pl.program_id(0)
    x = x_ref[...]
    cols = jax.lax.broadcasted_iota(jnp.int32, (n_rows, bc), 1) + j * bc
    xm = jnp.where(cols < n_cols, x, 0.0)
    s = jnp.sum(xm)
    t = tgt_ref[...]
    g = jnp.sum(jnp.where(cols == t, xm, 0.0))
    eps = _SMOOTHING / (n_cols - 1)
    contrib = (-eps) * s + (eps - _CONFIDENCE) * g

    @pl.when(j == 0)
    def _init():
        acc_ref[0] = 0.0

    acc_ref[0] += contrib

    @pl.when(j == nblk - 1)
    def _fin():
        out_ref[0, 0] = acc_ref[0] / n_rows


def kernel(pred, target):
    n_rows, n_cols = pred.shape
    bc = min(2048, n_cols)
    nblk = pl.cdiv(n_cols, bc)
    tgt2d = target.astype(jnp.int32).reshape(n_rows, 1)
    out = pl.pallas_call(
        functools.partial(
            _nll_block, n_rows=n_rows, n_cols=n_cols, bc=bc, nblk=nblk
        ),
        grid=(nblk,),
        in_specs=[
            pl.BlockSpec((n_rows, 1), lambda j: (0, 0)),
            pl.BlockSpec((n_rows, bc), lambda j: (0, j)),
        ],
        out_specs=pl.BlockSpec(
            (1, 1), lambda j: (0, 0), memory_space=pltpu.SMEM
        ),
        out_shape=jax.ShapeDtypeStruct((1, 1), jnp.float32),
        scratch_shapes=[pltpu.SMEM((1,), jnp.float32)],
    )(tgt2d, pred)
    return out[0, 0]


# speedup vs baseline: 2.2707x; 1.0012x over previous
import functools

import jax
import jax.numpy as jnp
from jax.experimental import pallas as pl
from jax.experimental.pallas import tpu as pltpu

_SMOOTHING = 0.1
_CONFIDENCE = 1.0 - _SMOOTHING


def _nll_block(tgt_ref, x_ref, out_ref, acc_ref, *, n_rows, n_cols, bc, nblk):
    j = pl.program_id(0)
    x = x_ref[...]
    cols = jax.lax.broadcasted_iota(jnp.int32, (n_rows, bc), 1) + j * bc
    xm = jnp.where(cols < n_cols, x, 0.0)
    s = jnp.sum(xm)
    t = tgt_ref[...]
    g = jnp.sum(jnp.where(cols == t, xm, 0.0))
    eps = _SMOOTHING / (n_cols - 1)
    contrib = (-eps) * s + (eps - _CONFIDENCE) * g

    @pl.when(j == 0)
    def _init():
        acc_ref[0] = 0.0

    acc_ref[0] += contrib

    @pl.when(j == nblk - 1)
    def _fin():
        out_ref[0, 0] = acc_ref[0] / n_rows


def kernel(pred, target):
    n_rows, n_cols = pred.shape
    bc = min(4096, n_cols)
    nblk = pl.cdiv(n_cols, bc)
    tgt2d = target.astype(jnp.int32).reshape(n_rows, 1)
    out = pl.pallas_call(
        functools.partial(
            _nll_block, n_rows=n_rows, n_cols=n_cols, bc=bc, nblk=nblk
        ),
        grid=(nblk,),
        in_specs=[
            pl.BlockSpec((n_rows, 1), lambda j: (0, 0)),
            pl.BlockSpec((n_rows, bc), lambda j: (0, j)),
        ],
        out_specs=pl.BlockSpec(
            (1, 1), lambda j: (0, 0), memory_space=pltpu.SMEM
        ),
        out_shape=jax.ShapeDtypeStruct((1, 1), jnp.float32),
        scratch_shapes=[pltpu.SMEM((1,), jnp.float32)],
    )(tgt2d, pred)
    return out[0, 0]


# speedup vs baseline: 7.4405x; 3.2768x over previous
import functools

import jax
import jax.numpy as jnp
from jax.experimental import pallas as pl
from jax.experimental.pallas import tpu as pltpu

_SMOOTHING = 0.1
_CONFIDENCE = 1.0 - _SMOOTHING


def _nll_block(tgt_ref, x_ref, out_ref, acc_ref, *, n_rows, n_cols, br, nblk):
    j = pl.program_id(0)
    x = x_ref[...]
    classes = jax.lax.broadcasted_iota(jnp.int32, (br, n_rows), 0) + j * br
    s = jnp.sum(x)
    t = tgt_ref[...]
    g = jnp.sum(jnp.where(classes == t, x, 0.0))
    eps = _SMOOTHING / (n_cols - 1)
    contrib = (-eps) * s + (eps - _CONFIDENCE) * g

    @pl.when(j == 0)
    def _init():
        acc_ref[0] = 0.0

    acc_ref[0] += contrib

    @pl.when(j == nblk - 1)
    def _fin():
        out_ref[0, 0] = acc_ref[0] / n_rows


def kernel(pred, target):
    n_rows, n_cols = pred.shape
    pred_t = pred.T
    br = 2000
    while n_cols % br:
        br //= 2
    nblk = n_cols // br
    tgt2d = target.astype(jnp.int32).reshape(1, n_rows)
    out = pl.pallas_call(
        functools.partial(
            _nll_block, n_rows=n_rows, n_cols=n_cols, br=br, nblk=nblk
        ),
        grid=(nblk,),
        in_specs=[
            pl.BlockSpec((1, n_rows), lambda j: (0, 0)),
            pl.BlockSpec((br, n_rows), lambda j: (j, 0)),
        ],
        out_specs=pl.BlockSpec(
            (1, 1), lambda j: (0, 0), memory_space=pltpu.SMEM
        ),
        out_shape=jax.ShapeDtypeStruct((1, 1), jnp.float32),
        scratch_shapes=[pltpu.SMEM((1,), jnp.float32)],
    )(tgt2d, pred_t)
    return out[0, 0]


# speedup vs baseline: 7.9125x; 1.0634x over previous
import functools

import jax
import jax.numpy as jnp
from jax.experimental import pallas as pl
from jax.experimental.pallas import tpu as pltpu

_SMOOTHING = 0.1
_CONFIDENCE = 1.0 - _SMOOTHING


def _nll_block(tgt_ref, x_ref, out_ref, acc_ref, *, n_rows, n_cols, br, nblk):
    j = pl.program_id(0)
    x = x_ref[...]
    classes = jax.lax.broadcasted_iota(jnp.int32, (br, n_rows), 0) + j * br
    t = tgt_ref[...]
    eps = _SMOOTHING / (n_cols - 1)
    w = jnp.where(classes == t, -_CONFIDENCE, -eps)
    contrib = jnp.sum(x * w)

    @pl.when(j == 0)
    def _init():
        acc_ref[0] = 0.0

    acc_ref[0] += contrib

    @pl.when(j == nblk - 1)
    def _fin():
        out_ref[0, 0] = acc_ref[0] / n_rows


def kernel(pred, target):
    n_rows, n_cols = pred.shape
    pred_t = pred.T
    br = 2000
    while n_cols % br:
        br //= 2
    nblk = n_cols // br
    tgt2d = target.astype(jnp.int32).reshape(1, n_rows)
    out = pl.pallas_call(
        functools.partial(
            _nll_block, n_rows=n_rows, n_cols=n_cols, br=br, nblk=nblk
        ),
        grid=(nblk,),
        in_specs=[
            pl.BlockSpec((1, n_rows), lambda j: (0, 0)),
            pl.BlockSpec((br, n_rows), lambda j: (j, 0)),
        ],
        out_specs=pl.BlockSpec(
            (1, 1), lambda j: (0, 0), memory_space=pltpu.SMEM
        ),
        out_shape=jax.ShapeDtypeStruct((1, 1), jnp.float32),
        scratch_shapes=[pltpu.SMEM((1,), jnp.float32)],
    )(tgt2d, pred_t)
    return out[0, 0]


# speedup vs baseline: 8.7499x; 1.1058x over previous
import functools

import jax
import jax.numpy as jnp
from jax.experimental import pallas as pl
from jax.experimental.pallas import tpu as pltpu

_SMOOTHING = 0.1
_CONFIDENCE = 1.0 - _SMOOTHING


def _nll_block(tgt_ref, x_ref, out_ref, acc_ref, *, n_rows, n_cols, br, nblk):
    j = pl.program_id(0)
    x = x_ref[...]
    classes = jax.lax.broadcasted_iota(jnp.int32, (br, n_rows), 0) + j * br
    t = tgt_ref[...]
    eps = _SMOOTHING / (n_cols - 1)
    w = jnp.where(classes == t, -_CONFIDENCE, -eps)
    contrib = jnp.sum(x * w)

    @pl.when(j == 0)
    def _init():
        acc_ref[0] = 0.0

    acc_ref[0] += contrib

    @pl.when(j == nblk - 1)
    def _fin():
        out_ref[0, 0] = acc_ref[0] / n_rows


def kernel(pred, target):
    n_rows, n_cols = pred.shape
    pred_t = pred.T
    br = 4000
    while n_cols % br:
        br //= 2
    nblk = n_cols // br
    tgt2d = target.astype(jnp.int32).reshape(1, n_rows)
    out = pl.pallas_call(
        functools.partial(
            _nll_block, n_rows=n_rows, n_cols=n_cols, br=br, nblk=nblk
        ),
        grid=(nblk,),
        in_specs=[
            pl.BlockSpec((1, n_rows), lambda j: (0, 0)),
            pl.BlockSpec((br, n_rows), lambda j: (j, 0)),
        ],
        out_specs=pl.BlockSpec(
            (1, 1), lambda j: (0, 0), memory_space=pltpu.SMEM
        ),
        out_shape=jax.ShapeDtypeStruct((1, 1), jnp.float32),
        scratch_shapes=[pltpu.SMEM((1,), jnp.float32)],
    )(tgt2d, pred_t)
    return out[0, 0]


# speedup vs baseline: 8.8141x; 1.0073x over previous
import functools

import jax
import jax.numpy as jnp
from jax.experimental import pallas as pl
from jax.experimental.pallas import tpu as pltpu

_SMOOTHING = 0.1
_CONFIDENCE = 1.0 - _SMOOTHING


def _nll_block(tgt_ref, x_ref, out_ref, acc_ref, *, n_rows, n_cols, br, nblk):
    j = pl.program_id(0)
    x = x_ref[...]
    classes = jax.lax.broadcasted_iota(jnp.int32, (br, n_rows), 0) + j * br
    t = tgt_ref[...]
    eps = _SMOOTHING / (n_cols - 1)
    w = jnp.where(classes == t, -_CONFIDENCE, -eps)
    contrib = jnp.sum(x * w)

    @pl.when(j == 0)
    def _init():
        acc_ref[0] = 0.0

    acc_ref[0] += contrib

    @pl.when(j == nblk - 1)
    def _fin():
        out_ref[0, 0] = acc_ref[0] / n_rows


def kernel(pred, target):
    n_rows, n_cols = pred.shape
    pred_t = pred.T
    br = 5000
    while n_cols % br:
        br //= 2
    nblk = n_cols // br
    tgt2d = target.astype(jnp.int32).reshape(1, n_rows)
    out = pl.pallas_call(
        functools.partial(
            _nll_block, n_rows=n_rows, n_cols=n_cols, br=br, nblk=nblk
        ),
        grid=(nblk,),
        in_specs=[
            pl.BlockSpec((1, n_rows), lambda j: (0, 0)),
            pl.BlockSpec((br, n_rows), lambda j: (j, 0)),
        ],
        out_specs=pl.BlockSpec(
            (1, 1), lambda j: (0, 0), memory_space=pltpu.SMEM
        ),
        out_shape=jax.ShapeDtypeStruct((1, 1), jnp.float32),
        scratch_shapes=[pltpu.SMEM((1,), jnp.float32)],
    )(tgt2d, pred_t)
    return out[0, 0]
